# baseline (speedup 1.0000x reference)
import functools

import jax
import jax.numpy as jnp
from jax import lax
from jax.experimental import pallas as pl
from jax.experimental.pallas import tpu as pltpu
from jax.experimental.pallas import tpu_sc as plsc

NE = 64
TOPK = 2
H = 2048
F = 768
T = 8192
R = T * TOPK


_CH = 512
_NCH = R // _CH
_TCH = _CH // TOPK


def _route_body(logits_ref, w_ref, p_ref, bid_ref, eid_ref, offs_ref):
    lg = logits_ref[...]
    col = lax.broadcasted_iota(jnp.int32, (T, NE), 1)
    m1 = jnp.max(lg, axis=1, keepdims=True)
    i1 = jnp.min(jnp.where(lg == m1, col, NE), axis=1, keepdims=True)
    lg2 = jnp.where(col == i1, -jnp.inf, lg)
    m2 = jnp.max(lg2, axis=1, keepdims=True)
    i2 = jnp.min(jnp.where(lg2 == m2, col, NE), axis=1, keepdims=True)
    z = jnp.exp(m2 - m1)
    w1 = 1.0 / (1.0 + z)
    w_ref[...] = jnp.concatenate([w1, 1.0 - w1], axis=1)

    o1 = (col == i1).astype(jnp.float32)
    o2 = (col == i2).astype(jnp.float32)
    cnts = jnp.sum(o1 + o2, axis=0, keepdims=True)

    r64 = lax.broadcasted_iota(jnp.int32, (NE, NE), 0)
    c64 = lax.broadcasted_iota(jnp.int32, (NE, NE), 1)
    offs = jnp.dot(cnts, (r64 < c64).astype(jnp.float32),
                   preferred_element_type=jnp.float32,
                   precision=lax.Precision.HIGHEST)

    rc = lax.broadcasted_iota(jnp.int32, (_CH, _CH), 0)
    cc = lax.broadcasted_iota(jnp.int32, (_CH, _CH), 1)
    tri = (rc >= cc).astype(jnp.float32)
    offs_i = offs.astype(jnp.int32)
    cnt_i = cnts.astype(jnp.int32)
    ends_i = offs_i + cnt_i
    offs_ref[...] = jnp.concatenate(
        [jnp.zeros((1, 1), jnp.int32), ends_i], axis=1)
    nonempty = cnt_i > 0
    sb = offs_i // _BM
    eb = (ends_i - 1) // _BM
    items = jnp.where(nonempty, eb - sb + 1, 0).astype(jnp.float32)
    ccum_it = jnp.dot(items, (r64 <= c64).astype(jnp.float32),
                      preferred_element_type=jnp.float32,
                      precision=lax.Precision.HIGHEST)
    excl_it = ccum_it - items
    ii = lax.broadcasted_iota(jnp.int32, (_GSTEPS, NE), 0)
    ee = lax.broadcasted_iota(jnp.int32, (_GSTEPS, NE), 1)
    eid = jnp.sum((ccum_it.astype(jnp.int32) <= ii).astype(jnp.int32),
                  axis=1)
    valid = eid < NE
    eidc = jnp.minimum(eid, NE - 1)
    oh = (ee == eidc[:, None]).astype(jnp.float32)
    sbg = jnp.sum(oh * sb.astype(jnp.float32), axis=1).astype(jnp.int32)
    exg = jnp.sum(oh * excl_it, axis=1).astype(jnp.int32)
    i_lin = lax.iota(jnp.int32, _GSTEPS)
    bid_ref[...] = jnp.where(valid, sbg + i_lin - exg,
                             _NB - 1)[None, :].astype(jnp.int32)
    eid_ref[...] = jnp.where(valid, eidc, NE - 1)[None, :].astype(jnp.int32)

    eidx = jnp.concatenate([i1, i2], axis=1)
    carry = jnp.zeros((1, NE), jnp.float32)
    for c in range(_NCH):
        ec = eidx[c * _TCH:(c + 1) * _TCH, :]
        e3 = lax.broadcasted_iota(jnp.int32, (_TCH, TOPK, NE), 2)
        oc = (ec[:, :, None] == e3).astype(jnp.float32).reshape(_CH, NE)
        inc = jnp.dot(tri, oc, preferred_element_type=jnp.float32,
                      precision=lax.Precision.HIGHEST)
        pos = jnp.sum(oc * (inc - 1.0 + carry + offs), axis=1)
        p_ref[c, :] = pos.astype(jnp.int32)
        carry = carry + jnp.sum(oc, axis=0, keepdims=True)


def _route(router_logits):
    return pl.pallas_call(
        _route_body,
        out_shape=(
            jax.ShapeDtypeStruct((T, TOPK), jnp.float32),
            jax.ShapeDtypeStruct((_NCH, _CH), jnp.int32),
            jax.ShapeDtypeStruct((1, _GSTEPS), jnp.int32),
            jax.ShapeDtypeStruct((1, _GSTEPS), jnp.int32),
            jax.ShapeDtypeStruct((1, NE + 1), jnp.int32),
        ),
    )(router_logits)


_NC = 2
_NSUB = 16
_NW = _NC * _NSUB
_RW = R // _NW
_CK = 16
_NIT = _RW // _CK


def _sc_dispatch(hidden, p_flat, tok_flat):
    mesh = plsc.VectorSubcoreMesh(core_axis_name="c", subcore_axis_name="s")

    @functools.partial(
        pl.kernel,
        out_type=jax.ShapeDtypeStruct((R, H), jnp.float32),
        mesh=mesh,
        scratch_types=[
            pltpu.VMEM((_CK,), jnp.int32),
            pltpu.VMEM((_CK,), jnp.int32),
            pltpu.VMEM((_CK,), jnp.int32),
            pltpu.VMEM((_CK,), jnp.int32),
            pltpu.VMEM((_CK, H), jnp.float32),
            pltpu.VMEM((_CK, H), jnp.float32),
            pltpu.SemaphoreType.DMA,
            pltpu.SemaphoreType.DMA,
            pltpu.SemaphoreType.DMA,
            pltpu.SemaphoreType.DMA,
        ],
    )
    def k(hid_hbm, p_hbm, tok_hbm, xs_hbm,
          tidx0, tidx1, pidx0, pidx1, buf0, buf1, gs0, gs1, ss0, ss1):
        wid = lax.axis_index("s") * _NC + lax.axis_index("c")
        base = wid * _RW
        tidx = (tidx0, tidx1)
        pidx = (pidx0, pidx1)
        buf = (buf0, buf1)
        gsem = (gs0, gs1)
        ssem = (ss0, ss1)

        pltpu.sync_copy(tok_hbm.at[pl.ds(base, _CK)], tidx[0])
        pltpu.sync_copy(p_hbm.at[pl.ds(base, _CK)], pidx[0])
        gat = [pltpu.async_copy(hid_hbm.at[tidx[0]], buf[0], gsem[0]), None]
        scat = [None, None]
        for j in range(_NIT):
            cur = j & 1
            nxt = cur ^ 1
            if j + 1 < _NIT:
                r1 = base + (j + 1) * _CK
                pltpu.sync_copy(tok_hbm.at[pl.ds(r1, _CK)], tidx[nxt])
                pltpu.sync_copy(p_hbm.at[pl.ds(r1, _CK)], pidx[nxt])
                if scat[nxt] is not None:
                    scat[nxt].wait()
                gat[nxt] = pltpu.async_copy(hid_hbm.at[tidx[nxt]], buf[nxt],
                                            gsem[nxt])
            gat[cur].wait()
            scat[cur] = pltpu.async_copy(buf[cur], xs_hbm.at[pidx[cur]],
                                         ssem[cur])
        scat[0].wait()
        scat[1].wait()

    return k(hidden, p_flat, tok_flat)


def _sc_combine_gather(ys, p_flat):
    mesh = plsc.VectorSubcoreMesh(core_axis_name="c", subcore_axis_name="s")

    @functools.partial(
        pl.kernel,
        out_type=jax.ShapeDtypeStruct((R, H), jnp.float32),
        mesh=mesh,
        scratch_types=[
            pltpu.VMEM((_CK,), jnp.int32),
            pltpu.VMEM((_CK,), jnp.int32),
            pltpu.VMEM((_CK, H), jnp.float32),
            pltpu.VMEM((_CK, H), jnp.float32),
            pltpu.SemaphoreType.DMA,
            pltpu.SemaphoreType.DMA,
            pltpu.SemaphoreType.DMA,
            pltpu.SemaphoreType.DMA,
        ],
    )
    def k(ys_hbm, p_hbm, oe_hbm,
          pidx0, pidx1, buf0, buf1, gs0, gs1, ws0, ws1):
        wid = lax.axis_index("s") * _NC + lax.axis_index("c")
        base = wid * _RW
        pidx = (pidx0, pidx1)
        buf = (buf0, buf1)
        gsem = (gs0, gs1)
        wsem = (ws0, ws1)

        pltpu.sync_copy(p_hbm.at[pl.ds(base, _CK)], pidx[0])
        gat = [pltpu.async_copy(ys_hbm.at[pidx[0]], buf[0], gsem[0]), None]
        wr = [None, None]
        for j in range(_NIT):
            cur = j & 1
            nxt = cur ^ 1
            if j + 1 < _NIT:
                r1 = base + (j + 1) * _CK
                pltpu.sync_copy(p_hbm.at[pl.ds(r1, _CK)], pidx[nxt])
                if wr[nxt] is not None:
                    wr[nxt].wait()
                gat[nxt] = pltpu.async_copy(ys_hbm.at[pidx[nxt]], buf[nxt],
                                            gsem[nxt])
            gat[cur].wait()
            r0 = base + j * _CK
            wr[cur] = pltpu.async_copy(buf[cur], oe_hbm.at[pl.ds(r0, _CK)],
                                       wsem[cur])
        wr[0].wait()
        wr[1].wait()

    return k(ys, p_flat)


_BM = 256
_NB = R // _BM
_GSTEPS = _NB + NE - 1


def _mlp_body(bid_ref, eid_ref, offs_ref, x_ref, g_ref, u_ref, d_ref, o_ref):
    i = pl.program_id(0)
    e = eid_ref[i]
    b = bid_ref[i]
    start = offs_ref[e]
    end = offs_ref[e + 1]
    row = b * _BM + lax.broadcasted_iota(jnp.int32, (_BM, 1), 0)
    mask = (row >= start) & (row < end)
    x = x_ref[...]
    g = jnp.dot(x, g_ref[0], preferred_element_type=jnp.float32)
    u = jnp.dot(x, u_ref[0], preferred_element_type=jnp.float32)
    h = g * lax.logistic(g) * u
    y = jnp.dot(h, d_ref[0], preferred_element_type=jnp.float32)
    o_ref[...] = jnp.where(mask, y, o_ref[...])


def _grouped_mlp(xs, gate_proj, up_proj, down_proj, bid, eid, offs):
    grid_spec = pltpu.PrefetchScalarGridSpec(
        num_scalar_prefetch=3,
        grid=(_GSTEPS,),
        in_specs=[
            pl.BlockSpec((_BM, H), lambda i, bid, eid, offs: (bid[i], 0)),
            pl.BlockSpec((1, H, F), lambda i, bid, eid, offs: (eid[i], 0, 0)),
            pl.BlockSpec((1, H, F), lambda i, bid, eid, offs: (eid[i], 0, 0)),
            pl.BlockSpec((1, F, H), lambda i, bid, eid, offs: (eid[i], 0, 0)),
        ],
        out_specs=pl.BlockSpec((_BM, H), lambda i, bid, eid, offs: (bid[i], 0)),
    )
    return pl.pallas_call(
        _mlp_body,
        grid_spec=grid_spec,
        out_shape=jax.ShapeDtypeStruct((R, H), jnp.float32),
    )(bid, eid, offs, xs, gate_proj, up_proj, down_proj)


_BT = 512


def _combine_body(w_ref, oe_ref, o_ref):
    w = w_ref[...]

    o_ref[...] = w[:, 0:1] * oe_ref[:, 0, :] + w[:, 1:2] * oe_ref[:, 1, :]


def _combine(w_pair, oe3):
    return pl.pallas_call(
        _combine_body,
        grid=(T // _BT,),
        in_specs=[
            pl.BlockSpec((_BT, TOPK), lambda i: (i, 0)),
            pl.BlockSpec((_BT, TOPK, H), lambda i: (i, 0, 0)),
        ],
        out_specs=pl.BlockSpec((_BT, H), lambda i: (i, 0)),
        out_shape=jax.ShapeDtypeStruct((T, H), jnp.float32),
    )(w_pair, oe3)


def kernel(hidden_states, router_logits, gate_proj, up_proj, down_proj):
    w_pair, p2d, bid2d, eid2d, offs2d = _route(router_logits)
    p_flat = p2d.reshape(R)
    bid = bid2d.reshape(_GSTEPS)
    eid = eid2d.reshape(_GSTEPS)
    offs = offs2d.reshape(NE + 1)
    tok_flat = jnp.arange(R, dtype=jnp.int32) // TOPK
    xs = _sc_dispatch(hidden_states, p_flat, tok_flat)
    ys = _grouped_mlp(xs, gate_proj, up_proj, down_proj, bid, eid, offs)
    oe = _sc_combine_gather(ys, p_flat)
    return _combine(w_pair, oe.reshape(T, TOPK, H))


# speedup vs baseline: 1.2334x; 1.2334x over previous
import functools

import jax
import jax.numpy as jnp
from jax import lax
from jax.experimental import pallas as pl
from jax.experimental.pallas import tpu as pltpu
from jax.experimental.pallas import tpu_sc as plsc

NE = 64
TOPK = 2
H = 2048
F = 768
T = 8192
R = T * TOPK


_CH = 512
_NCH = R // _CH
_TCH = _CH // TOPK


def _route_body(logits_ref, w_ref, p0_ref, p1_ref, bid_ref, eid_ref, offs_ref):
    lg = logits_ref[...]
    col = lax.broadcasted_iota(jnp.int32, (T, NE), 1)
    m1 = jnp.max(lg, axis=1, keepdims=True)
    i1 = jnp.min(jnp.where(lg == m1, col, NE), axis=1, keepdims=True)
    lg2 = jnp.where(col == i1, -jnp.inf, lg)
    m2 = jnp.max(lg2, axis=1, keepdims=True)
    i2 = jnp.min(jnp.where(lg2 == m2, col, NE), axis=1, keepdims=True)
    z = jnp.exp(m2 - m1)
    w1 = 1.0 / (1.0 + z)
    w_ref[...] = jnp.concatenate([w1, 1.0 - w1], axis=1)

    o1 = (col == i1).astype(jnp.float32)
    o2 = (col == i2).astype(jnp.float32)
    cnts = jnp.sum(o1 + o2, axis=0, keepdims=True)

    r64 = lax.broadcasted_iota(jnp.int32, (NE, NE), 0)
    c64 = lax.broadcasted_iota(jnp.int32, (NE, NE), 1)
    offs = jnp.dot(cnts, (r64 < c64).astype(jnp.float32),
                   preferred_element_type=jnp.float32,
                   precision=lax.Precision.HIGHEST)

    rc = lax.broadcasted_iota(jnp.int32, (_CH, _CH), 0)
    cc = lax.broadcasted_iota(jnp.int32, (_CH, _CH), 1)
    tri = (rc >= cc).astype(jnp.float32)
    offs_i = offs.astype(jnp.int32)
    cnt_i = cnts.astype(jnp.int32)
    ends_i = offs_i + cnt_i
    offs_ref[...] = jnp.concatenate(
        [jnp.zeros((1, 1), jnp.int32), ends_i], axis=1)
    nonempty = cnt_i > 0
    sb = offs_i // _BM
    eb = (ends_i - 1) // _BM
    items = jnp.where(nonempty, eb - sb + 1, 0).astype(jnp.float32)
    ccum_it = jnp.dot(items, (r64 <= c64).astype(jnp.float32),
                      preferred_element_type=jnp.float32,
                      precision=lax.Precision.HIGHEST)
    excl_it = ccum_it - items
    ii = lax.broadcasted_iota(jnp.int32, (_GSTEPS, NE), 0)
    ee = lax.broadcasted_iota(jnp.int32, (_GSTEPS, NE), 1)
    eid = jnp.sum((ccum_it.astype(jnp.int32) <= ii).astype(jnp.int32),
                  axis=1)
    valid = eid < NE
    eidc = jnp.minimum(eid, NE - 1)
    oh = (ee == eidc[:, None]).astype(jnp.float32)
    sbg = jnp.sum(oh * sb.astype(jnp.float32), axis=1).astype(jnp.int32)
    exg = jnp.sum(oh * excl_it, axis=1).astype(jnp.int32)
    i_lin = lax.iota(jnp.int32, _GSTEPS)
    bid_ref[...] = jnp.where(valid, sbg + i_lin - exg,
                             _NB - 1)[None, :].astype(jnp.int32)
    eid_ref[...] = jnp.where(valid, eidc, NE - 1)[None, :].astype(jnp.int32)

    eidx = jnp.concatenate([i1, i2], axis=1)
    ones_col = jnp.ones((NE, 1), jnp.float32)
    carry = jnp.zeros((1, NE), jnp.float32)
    for c in range(_NCH):
        ec = eidx[c * _TCH:(c + 1) * _TCH, :]
        e3 = lax.broadcasted_iota(jnp.int32, (_TCH, TOPK, NE), 2)
        oc = (ec[:, :, None] == e3).astype(jnp.float32).reshape(_CH, NE)
        inc = jnp.dot(tri, oc, preferred_element_type=jnp.float32,
                      precision=lax.Precision.HIGHEST)
        v3 = (oc * (inc - 1.0 + carry + offs)).reshape(_TCH, TOPK, NE)
        pos0 = jnp.dot(v3[:, 0, :], ones_col,
                       preferred_element_type=jnp.float32,
                       precision=lax.Precision.HIGHEST)
        pos1 = jnp.dot(v3[:, 1, :], ones_col,
                       preferred_element_type=jnp.float32,
                       precision=lax.Precision.HIGHEST)
        p0_ref[0:1, c * _TCH:(c + 1) * _TCH] = (
            pos0.astype(jnp.int32).reshape(1, _TCH))
        p1_ref[0:1, c * _TCH:(c + 1) * _TCH] = (
            pos1.astype(jnp.int32).reshape(1, _TCH))
        carry = carry + jnp.sum(oc, axis=0, keepdims=True)


def _route(router_logits):
    return pl.pallas_call(
        _route_body,
        out_shape=(
            jax.ShapeDtypeStruct((T, TOPK), jnp.float32),
            jax.ShapeDtypeStruct((1, T), jnp.int32),
            jax.ShapeDtypeStruct((1, T), jnp.int32),
            jax.ShapeDtypeStruct((1, _GSTEPS), jnp.int32),
            jax.ShapeDtypeStruct((1, _GSTEPS), jnp.int32),
            jax.ShapeDtypeStruct((1, NE + 1), jnp.int32),
        ),
    )(router_logits)


_NC = 2
_NSUB = 16
_NW = _NC * _NSUB
_RW = R // _NW
_CK = 16
_NIT = _RW // _CK


_TW = T // _NW


def _sc_dispatch(hidden, p0, p1):
    mesh = plsc.VectorSubcoreMesh(core_axis_name="c", subcore_axis_name="s")

    @functools.partial(
        pl.kernel,
        out_type=jax.ShapeDtypeStruct((R, H), jnp.float32),
        mesh=mesh,
        scratch_types=[
            pltpu.VMEM((_CK,), jnp.int32),
            pltpu.VMEM((_CK,), jnp.int32),
            pltpu.VMEM((_CK,), jnp.int32),
            pltpu.VMEM((_CK,), jnp.int32),
            pltpu.VMEM((_CK, H), jnp.float32),
            pltpu.VMEM((_CK, H), jnp.float32),
            pltpu.SemaphoreType.DMA,
            pltpu.SemaphoreType.DMA,
            pltpu.SemaphoreType.DMA,
            pltpu.SemaphoreType.DMA,
            pltpu.SemaphoreType.DMA,
            pltpu.SemaphoreType.DMA,
        ],
    )
    def k(hid_hbm, p0_hbm, p1_hbm, xs_hbm,
          i00, i01, i10, i11, buf0, buf1, ls0, ls1, sa0, sa1, sb0, sb1):
        wid = lax.axis_index("s") * _NC + lax.axis_index("c")
        base = wid * _TW
        i0 = (i00, i01)
        i1 = (i10, i11)
        buf = (buf0, buf1)
        lsem = (ls0, ls1)
        asem = (sa0, sa1)
        bsem = (sb0, sb1)
        nit = _TW // _CK

        pltpu.sync_copy(p0_hbm.at[pl.ds(base, _CK)], i0[0])
        pltpu.sync_copy(p1_hbm.at[pl.ds(base, _CK)], i1[0])
        rd = [pltpu.async_copy(hid_hbm.at[pl.ds(base, _CK)], buf[0], lsem[0]),
              None]
        sca = [None, None]
        scb = [None, None]
        for j in range(nit):
            cur = j & 1
            nxt = cur ^ 1
            if j + 1 < nit:
                t1 = base + (j + 1) * _CK
                pltpu.sync_copy(p0_hbm.at[pl.ds(t1, _CK)], i0[nxt])
                pltpu.sync_copy(p1_hbm.at[pl.ds(t1, _CK)], i1[nxt])
                if sca[nxt] is not None:
                    sca[nxt].wait()
                    scb[nxt].wait()
                rd[nxt] = pltpu.async_copy(hid_hbm.at[pl.ds(t1, _CK)],
                                           buf[nxt], lsem[nxt])
            rd[cur].wait()
            sca[cur] = pltpu.async_copy(buf[cur], xs_hbm.at[i0[cur]],
                                        asem[cur])
            scb[cur] = pltpu.async_copy(buf[cur], xs_hbm.at[i1[cur]],
                                        bsem[cur])
        sca[0].wait()
        scb[0].wait()
        sca[1].wait()
        scb[1].wait()

    return k(hidden, p0, p1)


def _sc_combine_gather(ys, p_flat):
    mesh = plsc.VectorSubcoreMesh(core_axis_name="c", subcore_axis_name="s")

    @functools.partial(
        pl.kernel,
        out_type=jax.ShapeDtypeStruct((R, H), jnp.float32),
        mesh=mesh,
        scratch_types=[
            pltpu.VMEM((_CK,), jnp.int32),
            pltpu.VMEM((_CK,), jnp.int32),
            pltpu.VMEM((_CK, H), jnp.float32),
            pltpu.VMEM((_CK, H), jnp.float32),
            pltpu.SemaphoreType.DMA,
            pltpu.SemaphoreType.DMA,
            pltpu.SemaphoreType.DMA,
            pltpu.SemaphoreType.DMA,
        ],
    )
    def k(ys_hbm, p_hbm, oe_hbm,
          pidx0, pidx1, buf0, buf1, gs0, gs1, ws0, ws1):
        wid = lax.axis_index("s") * _NC + lax.axis_index("c")
        base = wid * _RW
        pidx = (pidx0, pidx1)
        buf = (buf0, buf1)
        gsem = (gs0, gs1)
        wsem = (ws0, ws1)

        pltpu.sync_copy(p_hbm.at[pl.ds(base, _CK)], pidx[0])
        gat = [pltpu.async_copy(ys_hbm.at[pidx[0]], buf[0], gsem[0]), None]
        wr = [None, None]
        for j in range(_NIT):
            cur = j & 1
            nxt = cur ^ 1
            if j + 1 < _NIT:
                r1 = base + (j + 1) * _CK
                pltpu.sync_copy(p_hbm.at[pl.ds(r1, _CK)], pidx[nxt])
                if wr[nxt] is not None:
                    wr[nxt].wait()
                gat[nxt] = pltpu.async_copy(ys_hbm.at[pidx[nxt]], buf[nxt],
                                            gsem[nxt])
            gat[cur].wait()
            r0 = base + j * _CK
            wr[cur] = pltpu.async_copy(buf[cur], oe_hbm.at[pl.ds(r0, _CK)],
                                       wsem[cur])
        wr[0].wait()
        wr[1].wait()

    return k(ys, p_flat)


_BM = 512
_NB = R // _BM
_GSTEPS = _NB + NE - 1


def _mlp_body(bid_ref, eid_ref, offs_ref, x_ref, g_ref, u_ref, d_ref, o_ref):
    i = pl.program_id(0)
    e = eid_ref[i]
    b = bid_ref[i]
    start = offs_ref[e]
    end = offs_ref[e + 1]
    row = b * _BM + lax.broadcasted_iota(jnp.int32, (_BM, 1), 0)
    mask = (row >= start) & (row < end)
    x = x_ref[...]
    g = jnp.dot(x, g_ref[0], preferred_element_type=jnp.float32)
    u = jnp.dot(x, u_ref[0], preferred_element_type=jnp.float32)
    h = g * lax.logistic(g) * u
    y = jnp.dot(h, d_ref[0], preferred_element_type=jnp.float32)
    o_ref[...] = jnp.where(mask, y, o_ref[...])


def _grouped_mlp(xs, gate_proj, up_proj, down_proj, bid, eid, offs):
    grid_spec = pltpu.PrefetchScalarGridSpec(
        num_scalar_prefetch=3,
        grid=(_GSTEPS,),
        in_specs=[
            pl.BlockSpec((_BM, H), lambda i, bid, eid, offs: (bid[i], 0)),
            pl.BlockSpec((1, H, F), lambda i, bid, eid, offs: (eid[i], 0, 0)),
            pl.BlockSpec((1, H, F), lambda i, bid, eid, offs: (eid[i], 0, 0)),
            pl.BlockSpec((1, F, H), lambda i, bid, eid, offs: (eid[i], 0, 0)),
        ],
        out_specs=pl.BlockSpec((_BM, H), lambda i, bid, eid, offs: (bid[i], 0)),
    )
    return pl.pallas_call(
        _mlp_body,
        grid_spec=grid_spec,
        out_shape=jax.ShapeDtypeStruct((R, H), jnp.float32),
    )(bid, eid, offs, xs, gate_proj, up_proj, down_proj)


_BT = 512


def _combine_body(w_ref, oe_ref, o_ref):
    w = w_ref[...]

    o_ref[...] = w[:, 0:1] * oe_ref[0] + w[:, 1:2] * oe_ref[1]


def _combine(w_pair, oe3):
    return pl.pallas_call(
        _combine_body,
        grid=(T // _BT,),
        in_specs=[
            pl.BlockSpec((_BT, TOPK), lambda i: (i, 0)),
            pl.BlockSpec((TOPK, _BT, H), lambda i: (0, i, 0)),
        ],
        out_specs=pl.BlockSpec((_BT, H), lambda i: (i, 0)),
        out_shape=jax.ShapeDtypeStruct((T, H), jnp.float32),
    )(w_pair, oe3)


def kernel(hidden_states, router_logits, gate_proj, up_proj, down_proj):
    w_pair, p0_2d, p1_2d, bid2d, eid2d, offs2d = _route(router_logits)
    p0 = p0_2d.reshape(T)
    p1 = p1_2d.reshape(T)
    bid = bid2d.reshape(_GSTEPS)
    eid = eid2d.reshape(_GSTEPS)
    offs = offs2d.reshape(NE + 1)
    xs = _sc_dispatch(hidden_states, p0, p1)
    ys = _grouped_mlp(xs, gate_proj, up_proj, down_proj, bid, eid, offs)
    p_cat = jnp.concatenate([p0, p1])
    oe = _sc_combine_gather(ys, p_cat)
    return _combine(w_pair, oe.reshape(TOPK, T, H))
